# baseline (speedup 1.0000x reference)
import dataclasses
import functools

import jax
import jax.numpy as jnp
from jax import lax
from jax.experimental import pallas as pl
from jax.experimental.pallas import tpu as pltpu
from jax.experimental.pallas import tpu_sc as plsc

TOPK = 2

_SC_CORES = 2
_SC_SUBCORES = 16
_NW = _SC_CORES * _SC_SUBCORES


def _router_body(x_ref, rw_ref, probs_ref, w_ref, sd_ref, cnt_ref,
                 carry_ref, *, e, cap):
    tm = x_ref.shape[0]
    logits = lax.dot_general(
        x_ref[...], rw_ref[...], (((1,), (1,)), ((), ())),
        preferred_element_type=jnp.float32)
    m = jnp.max(logits, axis=-1, keepdims=True)
    ex = jnp.exp(logits - m)
    probs = ex / jnp.sum(ex, axis=-1, keepdims=True)
    probs_ref[...] = probs

    iota8 = lax.broadcasted_iota(jnp.int32, (tm, e), 1)
    m1 = jnp.max(probs, axis=-1, keepdims=True)
    i1 = jnp.min(jnp.where(probs == m1, iota8, e), axis=-1, keepdims=True)
    mask1 = iota8 == i1
    probs2 = jnp.where(mask1, -1.0, probs)
    m2 = jnp.max(probs2, axis=-1, keepdims=True)
    i2 = jnp.min(jnp.where(probs2 == m2, iota8, e), axis=-1, keepdims=True)
    mask2 = iota8 == i2
    s = m1 + m2 + 1e-10
    w1 = m1 / s
    w2 = m2 / s

    @pl.when(pl.program_id(0) == 0)
    def _():
        carry_ref[...] = jnp.zeros_like(carry_ref)

    cnt0 = carry_ref[...]
    h1 = mask1.astype(jnp.int32)
    h2 = mask2.astype(jnp.int32)
    htok = h1 + h2
    a = htok
    k = 1
    while k < tm:
        a = a + jnp.concatenate(
            [jnp.zeros((k, e), jnp.int32), a[:-k]], axis=0)
        k *= 2
    excl = a - htok + cnt0
    pos1 = jnp.sum(excl * h1, axis=-1, keepdims=True)
    pos2 = jnp.sum(excl * h2, axis=-1, keepdims=True)
    slot1 = i1 * cap + pos1
    slot2 = i2 * cap + pos2
    ok1 = pos1 < cap
    ok2 = pos2 < cap
    dump = e * cap
    sd_ref[...] = jnp.concatenate(
        [jnp.where(ok1, slot1, dump), jnp.where(ok2, slot2, dump)], axis=-1)
    w_ref[...] = jnp.concatenate(
        [jnp.where(ok1, w1, 0.0), jnp.where(ok2, w2, 0.0)], axis=-1)
    new_cnt = cnt0 + jnp.sum(htok, axis=0, keepdims=True)
    carry_ref[...] = new_cnt
    cnt_ref[...] = jnp.minimum(new_cnt, cap)


def _router(x2d, router_w, cap):
    n, c = x2d.shape
    e = router_w.shape[0]
    tm = 512
    body = functools.partial(_router_body, e=e, cap=cap)
    return pl.pallas_call(
        body,
        grid=(n // tm,),
        in_specs=[
            pl.BlockSpec((tm, c), lambda i: (i, 0)),
            pl.BlockSpec((e, c), lambda i: (0, 0)),
        ],
        out_specs=[
            pl.BlockSpec((tm, e), lambda i: (i, 0)),
            pl.BlockSpec((tm, TOPK), lambda i: (i, 0)),
            pl.BlockSpec((tm, TOPK), lambda i: (i, 0)),
            pl.BlockSpec((1, e), lambda i: (0, 0)),
        ],
        out_shape=[
            jax.ShapeDtypeStruct((n, e), jnp.float32),
            jax.ShapeDtypeStruct((n, TOPK), jnp.float32),
            jax.ShapeDtypeStruct((n, TOPK), jnp.int32),
            jax.ShapeDtypeStruct((1, e), jnp.int32),
        ],
        scratch_shapes=[pltpu.VMEM((1, e), jnp.int32)],
    )(x2d, router_w)


def _dispatch_sc(x2d, slot_a, slot_b, w_a, w_b, cap_rows):
    n, c = x2d.shape
    tok_per_w = n // _NW
    ch = 64
    mesh = plsc.VectorSubcoreMesh(core_axis_name="c", subcore_axis_name="s")
    cp = pltpu.CompilerParams()
    if "needs_layout_passes" in pltpu.CompilerParams.__dataclass_fields__:
        cp = dataclasses.replace(cp, needs_layout_passes=False)

    @functools.partial(
        pl.kernel, mesh=mesh,
        out_type=[
            jax.ShapeDtypeStruct((cap_rows + 8, c), jnp.float32),
            jax.ShapeDtypeStruct((cap_rows + 8, 128), jnp.float32),
        ],
        compiler_params=cp,
        scratch_types=[
            pltpu.VMEM((ch,), jnp.int32),
            pltpu.VMEM((ch,), jnp.int32),
            pltpu.VMEM((ch, c), jnp.float32),
            pltpu.VMEM((ch,), jnp.float32),
            pltpu.VMEM((ch,), jnp.float32),
            pltpu.VMEM((ch, 128), jnp.float32),
            pltpu.VMEM((ch, 128), jnp.float32),
            pltpu.SemaphoreType.DMA,
            pltpu.SemaphoreType.DMA,
            pltpu.SemaphoreType.DMA,
            pltpu.SemaphoreType.DMA,
        ],
    )
    def k(x_hbm, sa_hbm, sb_hbm, wa_hbm, wb_hbm, xg_hbm, ws_hbm,
          ia_v, ib_v, buf, wa_v, wb_v, ra_v, rb_v, sem_a, sem_b, sem_c, sem_d):
        wid = lax.axis_index("s") * _SC_CORES + lax.axis_index("c")
        base = wid * tok_per_w
        for ci in range(tok_per_w // ch):
            off = base + ci * ch
            pltpu.sync_copy(sa_hbm.at[pl.ds(off, ch)], ia_v)
            pltpu.sync_copy(sb_hbm.at[pl.ds(off, ch)], ib_v)
            pltpu.sync_copy(wa_hbm.at[pl.ds(off, ch)], wa_v)
            pltpu.sync_copy(wb_hbm.at[pl.ds(off, ch)], wb_v)
            pltpu.sync_copy(x_hbm.at[pl.ds(off, ch)], buf)

            @pl.loop(0, ch)
            def _(i):
                zi = lax.iota(jnp.int32, 16) * 0
                va = plsc.load_gather(wa_v, [zi + i])
                vb = plsc.load_gather(wb_v, [zi + i])
                ra_v[i, pl.ds(0, 16)] = va
                rb_v[i, pl.ds(0, 16)] = vb

            cp_a = pltpu.async_copy(buf, xg_hbm.at[ia_v], sem_a)
            cp_b = pltpu.async_copy(buf, xg_hbm.at[ib_v], sem_b)
            cp_c = pltpu.async_copy(ra_v, ws_hbm.at[ia_v], sem_c)
            cp_d = pltpu.async_copy(rb_v, ws_hbm.at[ib_v], sem_d)
            cp_a.wait()
            cp_b.wait()
            cp_c.wait()
            cp_d.wait()

    return k(x2d, slot_a, slot_b, w_a, w_b)


def _ffn_body(cnt_ref, xg_ref, ws_ref, fc_hbm, pj_hbm, eo_ref,
              fc_buf, pj_buf, sems, *, ne, tm, mt):
    t = pl.program_id(0)
    e = t // mt
    m = lax.rem(t, mt)

    def weights_dma(src_e, parity):
        nq = sems.shape[2]
        hq = fc_buf.shape[1] // nq
        cq = pj_buf.shape[1] // nq
        cps = []
        for q in range(nq):
            cps.append(pltpu.make_async_copy(
                fc_hbm.at[src_e, pl.ds(q * hq, hq)],
                fc_buf.at[parity, pl.ds(q * hq, hq)],
                sems.at[0, parity, q]))
            cps.append(pltpu.make_async_copy(
                pj_hbm.at[src_e, pl.ds(q * cq, cq)],
                pj_buf.at[parity, pl.ds(q * cq, cq)],
                sems.at[1, parity, q]))
        return cps

    @pl.when(t == 0)
    def _():
        for cp in weights_dma(0, 0):
            cp.start()

    @pl.when((m == 0) & (e < ne))
    def _():
        for cp in weights_dma(e, e % 2):
            cp.wait()

        @pl.when(e + 1 < ne)
        def _():
            for cp in weights_dma(e + 1, (e + 1) % 2):
                cp.start()

    @pl.when(t == ne * mt)
    def _():
        eo_ref[...] = jnp.zeros_like(eo_ref)

    @pl.when((e < ne) & (m * tm < cnt_ref[jnp.minimum(e, ne - 1)]))
    def _():
        p = e % 2
        h = lax.dot_general(
            xg_ref[...], fc_buf[p], (((1,), (1,)), ((), ())),
            preferred_element_type=jnp.float32)
        h = jnp.square(jnp.maximum(h, 0.0))
        eo_ref[...] = lax.dot_general(
            h, pj_buf[p], (((1,), (1,)), ((), ())),
            preferred_element_type=jnp.float32) * ws_ref[:, 0:1]


def _ffn(counts, xg, wslot, fc_w, proj_w, cap):
    e, hd, c = fc_w.shape
    tm = 256
    mt = cap // tm

    def io_idx(t, cnt):
        ei = t // mt
        mi = lax.rem(t, mt)
        ec = jnp.minimum(ei, e - 1)
        nm = jnp.maximum(lax.div(cnt[ec] + tm - 1, tm), 1)
        return (jnp.where(t < e * mt, ec * mt + jnp.minimum(mi, nm - 1), 0), 0)

    def out_idx(t, cnt):
        ei = t // mt
        mi = lax.rem(t, mt)
        ec = jnp.minimum(ei, e - 1)
        nm = jnp.maximum(lax.div(cnt[ec] + tm - 1, tm), 1)
        return (jnp.where(t < e * mt, ec * mt + jnp.minimum(mi, nm - 1), e * mt), 0)

    grid_spec = pltpu.PrefetchScalarGridSpec(
        num_scalar_prefetch=1,
        grid=(e * mt + 1,),
        in_specs=[
            pl.BlockSpec((tm, c), io_idx),
            pl.BlockSpec((tm, 128), io_idx),
            pl.BlockSpec(memory_space=pltpu.MemorySpace.HBM),
            pl.BlockSpec(memory_space=pltpu.MemorySpace.HBM),
        ],
        out_specs=pl.BlockSpec((tm, c), out_idx),
        scratch_shapes=[
            pltpu.VMEM((2, hd, c), jnp.float32),
            pltpu.VMEM((2, c, hd), jnp.float32),
            pltpu.SemaphoreType.DMA((2, 2, 4)),
        ],
    )
    body = functools.partial(_ffn_body, ne=e, tm=tm, mt=mt)
    return pl.pallas_call(
        body,
        grid_spec=grid_spec,
        out_shape=jax.ShapeDtypeStruct((e * cap + tm, c), jnp.float32),
    )(counts, xg, wslot, fc_w, proj_w)


def _combine_sc(eo, slot_flat):
    rows_, c = eo.shape
    ent = slot_flat.shape[0]
    n = ent // TOPK
    tok_per_w = n // _NW
    tch = 16
    nch = tok_per_w // tch
    mesh = plsc.VectorSubcoreMesh(core_axis_name="c", subcore_axis_name="s")

    @functools.partial(
        pl.kernel, mesh=mesh,
        out_type=jax.ShapeDtypeStruct((n, c), jnp.float32),
        scratch_types=[
            pltpu.VMEM((2, TOPK * tch), jnp.int32),
            pltpu.VMEM((2, TOPK * tch, c), jnp.float32),
            pltpu.VMEM((tch, c), jnp.float32),
            pltpu.SemaphoreType.DMA((2,)),
        ],
    )
    def k(eo_hbm, idx_hbm, o_hbm, idx_v, rows_v, out_v, sems):
        wid = lax.axis_index("s") * _SC_CORES + lax.axis_index("c")
        tbase = wid * tok_per_w

        def start_gather(ci, par):
            ebase = TOPK * (tbase + ci * tch)
            pltpu.sync_copy(idx_hbm.at[pl.ds(ebase, TOPK * tch)], idx_v.at[par])
            return pltpu.async_copy(eo_hbm.at[idx_v.at[par]], rows_v.at[par],
                                    sems.at[par])

        cps = [start_gather(0, 0), None]
        for ci in range(nch):
            par = ci % 2
            if ci + 1 < nch:
                cps[(ci + 1) % 2] = start_gather(ci + 1, (ci + 1) % 2)
            cps[par].wait()

            @pl.loop(0, tch)
            def _(ti):
                @pl.loop(0, c, step=64)
                def _(cc):
                    for u in range(4):
                        r0 = rows_v[par, 2 * ti, pl.ds(cc + u * 16, 16)]
                        r1 = rows_v[par, 2 * ti + 1, pl.ds(cc + u * 16, 16)]
                        out_v[ti, pl.ds(cc + u * 16, 16)] = r0 + r1

            pltpu.sync_copy(out_v, o_hbm.at[pl.ds(tbase + ci * tch, tch)])

    return k(eo, slot_flat)


def kernel(x, router_w, fc_w, proj_w):
    b, t, c = x.shape
    n = b * t
    e, hd, _ = fc_w.shape
    cap = 2 * n * TOPK // e

    x2d = x.reshape(n, c)
    probs, w2, slot_d, counts = _router(x2d, router_w, cap)
    xg, wslot = _dispatch_sc(x2d, slot_d[:, 0], slot_d[:, 1],
                             w2[:, 0], w2[:, 1], e * cap)
    eo = _ffn(counts.reshape(e), xg, wslot, fc_w, proj_w, cap)
    out = _combine_sc(eo, slot_d.reshape(-1))
    return out.reshape(b, t, c), probs.reshape(b, t, e)


# speedup vs baseline: 1.1168x; 1.1168x over previous
import dataclasses
import functools

import jax
import jax.numpy as jnp
from jax import lax
from jax.experimental import pallas as pl
from jax.experimental.pallas import tpu as pltpu
from jax.experimental.pallas import tpu_sc as plsc

TOPK = 2

_SC_CORES = 2
_SC_SUBCORES = 16
_NW = _SC_CORES * _SC_SUBCORES


def _router_body(x_ref, rw_ref, probs_ref, w_ref, sd_ref, cnt_ref,
                 carry_ref, *, e, cap):
    tm = x_ref.shape[0]
    logits = lax.dot_general(
        x_ref[...], rw_ref[...], (((1,), (1,)), ((), ())),
        preferred_element_type=jnp.float32)
    m = jnp.max(logits, axis=-1, keepdims=True)
    ex = jnp.exp(logits - m)
    probs = ex / jnp.sum(ex, axis=-1, keepdims=True)
    probs_ref[...] = probs

    iota8 = lax.broadcasted_iota(jnp.int32, (tm, e), 1)
    m1 = jnp.max(probs, axis=-1, keepdims=True)
    i1 = jnp.min(jnp.where(probs == m1, iota8, e), axis=-1, keepdims=True)
    mask1 = iota8 == i1
    probs2 = jnp.where(mask1, -1.0, probs)
    m2 = jnp.max(probs2, axis=-1, keepdims=True)
    i2 = jnp.min(jnp.where(probs2 == m2, iota8, e), axis=-1, keepdims=True)
    mask2 = iota8 == i2
    s = m1 + m2 + 1e-10
    w1 = m1 / s
    w2 = m2 / s

    @pl.when(pl.program_id(0) == 0)
    def _():
        carry_ref[...] = jnp.zeros_like(carry_ref)

    cnt0 = carry_ref[...]
    h1 = mask1.astype(jnp.int32)
    h2 = mask2.astype(jnp.int32)
    htok = h1 + h2
    a = htok
    k = 1
    while k < tm:
        a = a + jnp.concatenate(
            [jnp.zeros((k, e), jnp.int32), a[:-k]], axis=0)
        k *= 2
    excl = a - htok + cnt0
    pos1 = jnp.sum(excl * h1, axis=-1, keepdims=True)
    pos2 = jnp.sum(excl * h2, axis=-1, keepdims=True)
    slot1 = i1 * cap + pos1
    slot2 = i2 * cap + pos2
    ok1 = pos1 < cap
    ok2 = pos2 < cap
    dump = e * cap
    sd_ref[...] = jnp.concatenate(
        [jnp.where(ok1, slot1, dump), jnp.where(ok2, slot2, dump)], axis=-1)
    w_ref[...] = jnp.concatenate(
        [jnp.where(ok1, w1, 0.0), jnp.where(ok2, w2, 0.0)], axis=-1)
    new_cnt = cnt0 + jnp.sum(htok, axis=0, keepdims=True)
    carry_ref[...] = new_cnt
    cnt_ref[...] = jnp.minimum(new_cnt, cap)


def _router(x2d, router_w, cap):
    n, c = x2d.shape
    e = router_w.shape[0]
    tm = 512
    body = functools.partial(_router_body, e=e, cap=cap)
    return pl.pallas_call(
        body,
        grid=(n // tm,),
        in_specs=[
            pl.BlockSpec((tm, c), lambda i: (i, 0)),
            pl.BlockSpec((e, c), lambda i: (0, 0)),
        ],
        out_specs=[
            pl.BlockSpec((tm, e), lambda i: (i, 0)),
            pl.BlockSpec((tm, TOPK), lambda i: (i, 0)),
            pl.BlockSpec((tm, TOPK), lambda i: (i, 0)),
            pl.BlockSpec((1, e), lambda i: (0, 0)),
        ],
        out_shape=[
            jax.ShapeDtypeStruct((n, e), jnp.float32),
            jax.ShapeDtypeStruct((n, TOPK), jnp.float32),
            jax.ShapeDtypeStruct((n, TOPK), jnp.int32),
            jax.ShapeDtypeStruct((1, e), jnp.int32),
        ],
        scratch_shapes=[pltpu.VMEM((1, e), jnp.int32)],
    )(x2d, router_w)


def _dispatch_sc(x2d, slot_a, slot_b, w_a, w_b, cap_rows):
    n, c = x2d.shape
    tok_per_w = n // _NW
    ch = 64
    mesh = plsc.VectorSubcoreMesh(core_axis_name="c", subcore_axis_name="s")
    cp = pltpu.CompilerParams()
    if "needs_layout_passes" in pltpu.CompilerParams.__dataclass_fields__:
        cp = dataclasses.replace(cp, needs_layout_passes=False)

    @functools.partial(
        pl.kernel, mesh=mesh,
        out_type=[
            jax.ShapeDtypeStruct((cap_rows + 8, c), jnp.float32),
            jax.ShapeDtypeStruct((cap_rows + 8, 128), jnp.float32),
        ],
        compiler_params=cp,
        scratch_types=[
            pltpu.VMEM((ch,), jnp.int32),
            pltpu.VMEM((ch,), jnp.int32),
            pltpu.VMEM((ch, c), jnp.float32),
            pltpu.VMEM((ch,), jnp.float32),
            pltpu.VMEM((ch,), jnp.float32),
            pltpu.VMEM((ch, 128), jnp.float32),
            pltpu.VMEM((ch, 128), jnp.float32),
            pltpu.SemaphoreType.DMA,
            pltpu.SemaphoreType.DMA,
            pltpu.SemaphoreType.DMA,
            pltpu.SemaphoreType.DMA,
        ],
    )
    def k(x_hbm, sa_hbm, sb_hbm, wa_hbm, wb_hbm, xg_hbm, ws_hbm,
          ia_v, ib_v, buf, wa_v, wb_v, ra_v, rb_v, sem_a, sem_b, sem_c, sem_d):
        wid = lax.axis_index("s") * _SC_CORES + lax.axis_index("c")
        base = wid * tok_per_w
        for ci in range(tok_per_w // ch):
            off = base + ci * ch
            pltpu.sync_copy(sa_hbm.at[pl.ds(off, ch)], ia_v)
            pltpu.sync_copy(sb_hbm.at[pl.ds(off, ch)], ib_v)
            pltpu.sync_copy(wa_hbm.at[pl.ds(off, ch)], wa_v)
            pltpu.sync_copy(wb_hbm.at[pl.ds(off, ch)], wb_v)
            pltpu.sync_copy(x_hbm.at[pl.ds(off, ch)], buf)

            @pl.loop(0, ch)
            def _(i):
                zi = lax.iota(jnp.int32, 16) * 0
                va = plsc.load_gather(wa_v, [zi + i])
                vb = plsc.load_gather(wb_v, [zi + i])
                ra_v[i, pl.ds(0, 16)] = va
                rb_v[i, pl.ds(0, 16)] = vb

            cp_a = pltpu.async_copy(buf, xg_hbm.at[ia_v], sem_a)
            cp_b = pltpu.async_copy(buf, xg_hbm.at[ib_v], sem_b)
            cp_c = pltpu.async_copy(ra_v, ws_hbm.at[ia_v], sem_c)
            cp_d = pltpu.async_copy(rb_v, ws_hbm.at[ib_v], sem_d)
            cp_a.wait()
            cp_b.wait()
            cp_c.wait()
            cp_d.wait()

    return k(x2d, slot_a, slot_b, w_a, w_b)


def _ffn_body(cnt_ref, xg_ref, ws_ref, fc_hbm, pj_hbm, eo_ref,
              fc_buf, pj_buf, sems, *, ne, tm, mt):
    t = pl.program_id(0)
    e = t // mt
    m = lax.rem(t, mt)

    def weights_dma(src_e, parity):
        nq = sems.shape[2]
        hq = fc_buf.shape[1] // nq
        cq = pj_buf.shape[1] // nq
        cps = []
        for q in range(nq):
            cps.append(pltpu.make_async_copy(
                fc_hbm.at[src_e, pl.ds(q * hq, hq)],
                fc_buf.at[parity, pl.ds(q * hq, hq)],
                sems.at[0, parity, q]))
            cps.append(pltpu.make_async_copy(
                pj_hbm.at[src_e, pl.ds(q * cq, cq)],
                pj_buf.at[parity, pl.ds(q * cq, cq)],
                sems.at[1, parity, q]))
        return cps

    @pl.when(t == 0)
    def _():
        for cp in weights_dma(0, 0):
            cp.start()

    @pl.when((m == 0) & (e < ne))
    def _():
        for cp in weights_dma(e, e % 2):
            cp.wait()

        @pl.when(e + 1 < ne)
        def _():
            for cp in weights_dma(e + 1, (e + 1) % 2):
                cp.start()

    @pl.when(t == ne * mt)
    def _():
        eo_ref[...] = jnp.zeros_like(eo_ref)

    @pl.when((e < ne) & (m * tm < cnt_ref[jnp.minimum(e, ne - 1)]))
    def _():
        p = e % 2
        h = lax.dot_general(
            xg_ref[...], fc_buf[p], (((1,), (1,)), ((), ())),
            preferred_element_type=jnp.float32)
        h = jnp.square(jnp.maximum(h, 0.0))
        eo_ref[...] = lax.dot_general(
            h, pj_buf[p], (((1,), (1,)), ((), ())),
            preferred_element_type=jnp.float32) * ws_ref[:, 0:1]


def _ffn(counts, xg, wslot, fc_w, proj_w, cap):
    e, hd, c = fc_w.shape
    tm = 256
    mt = cap // tm

    def io_idx(t, cnt):
        ei = t // mt
        mi = lax.rem(t, mt)
        ec = jnp.minimum(ei, e - 1)
        nm = jnp.maximum(lax.div(cnt[ec] + tm - 1, tm), 1)
        return (jnp.where(t < e * mt, ec * mt + jnp.minimum(mi, nm - 1), 0), 0)

    def out_idx(t, cnt):
        ei = t // mt
        mi = lax.rem(t, mt)
        ec = jnp.minimum(ei, e - 1)
        nm = jnp.maximum(lax.div(cnt[ec] + tm - 1, tm), 1)
        return (jnp.where(t < e * mt, ec * mt + jnp.minimum(mi, nm - 1), e * mt), 0)

    grid_spec = pltpu.PrefetchScalarGridSpec(
        num_scalar_prefetch=1,
        grid=(e * mt + 1,),
        in_specs=[
            pl.BlockSpec((tm, c), io_idx),
            pl.BlockSpec((tm, 128), io_idx),
            pl.BlockSpec(memory_space=pltpu.MemorySpace.HBM),
            pl.BlockSpec(memory_space=pltpu.MemorySpace.HBM),
        ],
        out_specs=pl.BlockSpec((tm, c), out_idx),
        scratch_shapes=[
            pltpu.VMEM((2, hd, c), jnp.float32),
            pltpu.VMEM((2, c, hd), jnp.float32),
            pltpu.SemaphoreType.DMA((2, 2, 4)),
        ],
    )
    body = functools.partial(_ffn_body, ne=e, tm=tm, mt=mt)
    return pl.pallas_call(
        body,
        grid_spec=grid_spec,
        out_shape=jax.ShapeDtypeStruct((e * cap + tm, c), jnp.float32),
    )(counts, xg, wslot, fc_w, proj_w)


def _combine_sc(eo, slot_flat):
    rows_, c = eo.shape
    ent = slot_flat.shape[0]
    n = ent // TOPK
    tok_per_w = n // _NW
    tch = 16
    nch = tok_per_w // tch
    mesh = plsc.VectorSubcoreMesh(core_axis_name="c", subcore_axis_name="s")

    @functools.partial(
        pl.kernel, mesh=mesh,
        out_type=jax.ShapeDtypeStruct((n, c), jnp.float32),
        scratch_types=[
            pltpu.VMEM((2, TOPK * tch), jnp.int32),
            pltpu.VMEM((2, TOPK * tch, c), jnp.float32),
            pltpu.VMEM((tch, c), jnp.float32),
            pltpu.SemaphoreType.DMA((2,)),
        ],
    )
    def k(eo_hbm, idx_hbm, o_hbm, idx_v, rows_v, out_v, sems):
        wid = lax.axis_index("s") * _SC_CORES + lax.axis_index("c")
        tbase = wid * tok_per_w

        def start_gather(ci, par):
            ebase = TOPK * (tbase + ci * tch)
            pltpu.sync_copy(idx_hbm.at[pl.ds(ebase, TOPK * tch)], idx_v.at[par])
            return pltpu.async_copy(eo_hbm.at[idx_v.at[par]], rows_v.at[par],
                                    sems.at[par])

        cps = [start_gather(0, 0), None]
        for ci in range(nch):
            par = ci % 2
            if ci + 1 < nch:
                cps[(ci + 1) % 2] = start_gather(ci + 1, (ci + 1) % 2)
            cps[par].wait()

            @plsc.parallel_loop(0, tch)
            def _(ti):
                @plsc.parallel_loop(0, c, step=64)
                def _(cc):
                    for u in range(4):
                        r0 = rows_v[par, 2 * ti, pl.ds(cc + u * 16, 16)]
                        r1 = rows_v[par, 2 * ti + 1, pl.ds(cc + u * 16, 16)]
                        out_v[ti, pl.ds(cc + u * 16, 16)] = r0 + r1

            pltpu.sync_copy(out_v, o_hbm.at[pl.ds(tbase + ci * tch, tch)])

    return k(eo, slot_flat)


def kernel(x, router_w, fc_w, proj_w):
    b, t, c = x.shape
    n = b * t
    e, hd, _ = fc_w.shape
    cap = 2 * n * TOPK // e

    x2d = x.reshape(n, c)
    probs, w2, slot_d, counts = _router(x2d, router_w, cap)
    xg, wslot = _dispatch_sc(x2d, slot_d[:, 0], slot_d[:, 1],
                             w2[:, 0], w2[:, 1], e * cap)
    eo = _ffn(counts.reshape(e), xg, wslot, fc_w, proj_w, cap)
    out = _combine_sc(eo, slot_d.reshape(-1))
    return out.reshape(b, t, c), probs.reshape(b, t, e)


# speedup vs baseline: 1.1169x; 1.0001x over previous
import dataclasses
import functools

import jax
import jax.numpy as jnp
from jax import lax
from jax.experimental import pallas as pl
from jax.experimental.pallas import tpu as pltpu
from jax.experimental.pallas import tpu_sc as plsc

TOPK = 2

_SC_CORES = 2
_SC_SUBCORES = 16
_NW = _SC_CORES * _SC_SUBCORES


def _router_body(x_ref, rw_ref, probs_ref, w_ref, sd_ref, cnt_ref,
                 carry_ref, *, e, cap):
    tm = x_ref.shape[0]
    logits = lax.dot_general(
        x_ref[...], rw_ref[...], (((1,), (1,)), ((), ())),
        preferred_element_type=jnp.float32)
    m = jnp.max(logits, axis=-1, keepdims=True)
    ex = jnp.exp(logits - m)
    probs = ex / jnp.sum(ex, axis=-1, keepdims=True)
    probs_ref[...] = probs

    iota8 = lax.broadcasted_iota(jnp.int32, (tm, e), 1)
    m1 = jnp.max(probs, axis=-1, keepdims=True)
    i1 = jnp.min(jnp.where(probs == m1, iota8, e), axis=-1, keepdims=True)
    mask1 = iota8 == i1
    probs2 = jnp.where(mask1, -1.0, probs)
    m2 = jnp.max(probs2, axis=-1, keepdims=True)
    i2 = jnp.min(jnp.where(probs2 == m2, iota8, e), axis=-1, keepdims=True)
    mask2 = iota8 == i2
    s = m1 + m2 + 1e-10
    w1 = m1 / s
    w2 = m2 / s

    @pl.when(pl.program_id(0) == 0)
    def _():
        carry_ref[...] = jnp.zeros_like(carry_ref)

    cnt0 = carry_ref[...]
    h1 = mask1.astype(jnp.int32)
    h2 = mask2.astype(jnp.int32)
    htok = h1 + h2
    a = htok
    k = 1
    while k < tm:
        a = a + jnp.concatenate(
            [jnp.zeros((k, e), jnp.int32), a[:-k]], axis=0)
        k *= 2
    excl = a - htok + cnt0
    pos1 = jnp.sum(excl * h1, axis=-1, keepdims=True)
    pos2 = jnp.sum(excl * h2, axis=-1, keepdims=True)
    slot1 = i1 * cap + pos1
    slot2 = i2 * cap + pos2
    ok1 = pos1 < cap
    ok2 = pos2 < cap
    dump = e * cap
    sd_ref[...] = jnp.concatenate(
        [jnp.where(ok1, slot1, dump), jnp.where(ok2, slot2, dump)], axis=-1)
    w_ref[...] = jnp.concatenate(
        [jnp.where(ok1, w1, 0.0), jnp.where(ok2, w2, 0.0)], axis=-1)
    new_cnt = cnt0 + jnp.sum(htok, axis=0, keepdims=True)
    carry_ref[...] = new_cnt
    cnt_ref[...] = jnp.minimum(new_cnt, cap)


def _router(x2d, router_w, cap):
    n, c = x2d.shape
    e = router_w.shape[0]
    tm = 512
    body = functools.partial(_router_body, e=e, cap=cap)
    return pl.pallas_call(
        body,
        grid=(n // tm,),
        in_specs=[
            pl.BlockSpec((tm, c), lambda i: (i, 0)),
            pl.BlockSpec((e, c), lambda i: (0, 0)),
        ],
        out_specs=[
            pl.BlockSpec((tm, e), lambda i: (i, 0)),
            pl.BlockSpec((tm, TOPK), lambda i: (i, 0)),
            pl.BlockSpec((tm, TOPK), lambda i: (i, 0)),
            pl.BlockSpec((1, e), lambda i: (0, 0)),
        ],
        out_shape=[
            jax.ShapeDtypeStruct((n, e), jnp.float32),
            jax.ShapeDtypeStruct((n, TOPK), jnp.float32),
            jax.ShapeDtypeStruct((n, TOPK), jnp.int32),
            jax.ShapeDtypeStruct((1, e), jnp.int32),
        ],
        scratch_shapes=[pltpu.VMEM((1, e), jnp.int32)],
    )(x2d, router_w)


def _dispatch_sc(x2d, slot_a, slot_b, w_a, w_b, cap_rows):
    n, c = x2d.shape
    tok_per_w = n // _NW
    ch = 64
    mesh = plsc.VectorSubcoreMesh(core_axis_name="c", subcore_axis_name="s")
    cp = pltpu.CompilerParams()
    if "needs_layout_passes" in pltpu.CompilerParams.__dataclass_fields__:
        cp = dataclasses.replace(cp, needs_layout_passes=False)

    @functools.partial(
        pl.kernel, mesh=mesh,
        out_type=[
            jax.ShapeDtypeStruct((cap_rows + 8, c), jnp.float32),
            jax.ShapeDtypeStruct((cap_rows + 8, 128), jnp.float32),
        ],
        compiler_params=cp,
        scratch_types=[
            pltpu.VMEM((ch,), jnp.int32),
            pltpu.VMEM((ch,), jnp.int32),
            pltpu.VMEM((ch, c), jnp.float32),
            pltpu.VMEM((ch,), jnp.float32),
            pltpu.VMEM((ch,), jnp.float32),
            pltpu.VMEM((ch, 128), jnp.float32),
            pltpu.VMEM((ch, 128), jnp.float32),
            pltpu.SemaphoreType.DMA,
            pltpu.SemaphoreType.DMA,
            pltpu.SemaphoreType.DMA,
            pltpu.SemaphoreType.DMA,
        ],
    )
    def k(x_hbm, sa_hbm, sb_hbm, wa_hbm, wb_hbm, xg_hbm, ws_hbm,
          ia_v, ib_v, buf, wa_v, wb_v, ra_v, rb_v, sem_a, sem_b, sem_c, sem_d):
        wid = lax.axis_index("s") * _SC_CORES + lax.axis_index("c")
        base = wid * tok_per_w
        for ci in range(tok_per_w // ch):
            off = base + ci * ch
            pltpu.sync_copy(sa_hbm.at[pl.ds(off, ch)], ia_v)
            pltpu.sync_copy(sb_hbm.at[pl.ds(off, ch)], ib_v)
            pltpu.sync_copy(wa_hbm.at[pl.ds(off, ch)], wa_v)
            pltpu.sync_copy(wb_hbm.at[pl.ds(off, ch)], wb_v)
            pltpu.sync_copy(x_hbm.at[pl.ds(off, ch)], buf)

            @plsc.parallel_loop(0, ch)
            def _(i):
                zi = lax.iota(jnp.int32, 16) * 0
                va = plsc.load_gather(wa_v, [zi + i])
                vb = plsc.load_gather(wb_v, [zi + i])
                ra_v[i, pl.ds(0, 16)] = va
                rb_v[i, pl.ds(0, 16)] = vb

            cp_a = pltpu.async_copy(buf, xg_hbm.at[ia_v], sem_a)
            cp_b = pltpu.async_copy(buf, xg_hbm.at[ib_v], sem_b)
            cp_c = pltpu.async_copy(ra_v, ws_hbm.at[ia_v], sem_c)
            cp_d = pltpu.async_copy(rb_v, ws_hbm.at[ib_v], sem_d)
            cp_a.wait()
            cp_b.wait()
            cp_c.wait()
            cp_d.wait()

    return k(x2d, slot_a, slot_b, w_a, w_b)


def _ffn_body(cnt_ref, xg_ref, ws_ref, fc_hbm, pj_hbm, eo_ref,
              fc_buf, pj_buf, sems, *, ne, tm, mt):
    t = pl.program_id(0)
    e = t // mt
    m = lax.rem(t, mt)

    def weights_dma(src_e, parity):
        nq = sems.shape[2]
        hq = fc_buf.shape[1] // nq
        cq = pj_buf.shape[1] // nq
        cps = []
        for q in range(nq):
            cps.append(pltpu.make_async_copy(
                fc_hbm.at[src_e, pl.ds(q * hq, hq)],
                fc_buf.at[parity, pl.ds(q * hq, hq)],
                sems.at[0, parity, q]))
            cps.append(pltpu.make_async_copy(
                pj_hbm.at[src_e, pl.ds(q * cq, cq)],
                pj_buf.at[parity, pl.ds(q * cq, cq)],
                sems.at[1, parity, q]))
        return cps

    @pl.when(t == 0)
    def _():
        for cp in weights_dma(0, 0):
            cp.start()

    @pl.when((m == 0) & (e < ne))
    def _():
        for cp in weights_dma(e, e % 2):
            cp.wait()

        @pl.when(e + 1 < ne)
        def _():
            for cp in weights_dma(e + 1, (e + 1) % 2):
                cp.start()

    @pl.when(t == ne * mt)
    def _():
        eo_ref[...] = jnp.zeros_like(eo_ref)

    @pl.when((e < ne) & (m * tm < cnt_ref[jnp.minimum(e, ne - 1)]))
    def _():
        p = e % 2
        h = lax.dot_general(
            xg_ref[...], fc_buf[p], (((1,), (1,)), ((), ())),
            preferred_element_type=jnp.float32)
        h = jnp.square(jnp.maximum(h, 0.0))
        eo_ref[...] = lax.dot_general(
            h, pj_buf[p], (((1,), (1,)), ((), ())),
            preferred_element_type=jnp.float32) * ws_ref[:, 0:1]


def _ffn(counts, xg, wslot, fc_w, proj_w, cap):
    e, hd, c = fc_w.shape
    tm = 256
    mt = cap // tm

    def io_idx(t, cnt):
        ei = t // mt
        mi = lax.rem(t, mt)
        ec = jnp.minimum(ei, e - 1)
        nm = jnp.maximum(lax.div(cnt[ec] + tm - 1, tm), 1)
        return (jnp.where(t < e * mt, ec * mt + jnp.minimum(mi, nm - 1), 0), 0)

    def out_idx(t, cnt):
        ei = t // mt
        mi = lax.rem(t, mt)
        ec = jnp.minimum(ei, e - 1)
        nm = jnp.maximum(lax.div(cnt[ec] + tm - 1, tm), 1)
        return (jnp.where(t < e * mt, ec * mt + jnp.minimum(mi, nm - 1), e * mt), 0)

    grid_spec = pltpu.PrefetchScalarGridSpec(
        num_scalar_prefetch=1,
        grid=(e * mt + 1,),
        in_specs=[
            pl.BlockSpec((tm, c), io_idx),
            pl.BlockSpec((tm, 128), io_idx),
            pl.BlockSpec(memory_space=pltpu.MemorySpace.HBM),
            pl.BlockSpec(memory_space=pltpu.MemorySpace.HBM),
        ],
        out_specs=pl.BlockSpec((tm, c), out_idx),
        scratch_shapes=[
            pltpu.VMEM((2, hd, c), jnp.float32),
            pltpu.VMEM((2, c, hd), jnp.float32),
            pltpu.SemaphoreType.DMA((2, 2, 4)),
        ],
    )
    body = functools.partial(_ffn_body, ne=e, tm=tm, mt=mt)
    return pl.pallas_call(
        body,
        grid_spec=grid_spec,
        out_shape=jax.ShapeDtypeStruct((e * cap + tm, c), jnp.float32),
    )(counts, xg, wslot, fc_w, proj_w)


def _combine_sc(eo, slot_flat):
    rows_, c = eo.shape
    ent = slot_flat.shape[0]
    n = ent // TOPK
    tok_per_w = n // _NW
    tch = 16
    nch = tok_per_w // tch
    mesh = plsc.VectorSubcoreMesh(core_axis_name="c", subcore_axis_name="s")

    @functools.partial(
        pl.kernel, mesh=mesh,
        out_type=jax.ShapeDtypeStruct((n, c), jnp.float32),
        scratch_types=[
            pltpu.VMEM((2, TOPK * tch), jnp.int32),
            pltpu.VMEM((2, TOPK * tch, c), jnp.float32),
            pltpu.VMEM((tch, c), jnp.float32),
            pltpu.SemaphoreType.DMA((2,)),
        ],
    )
    def k(eo_hbm, idx_hbm, o_hbm, idx_v, rows_v, out_v, sems):
        wid = lax.axis_index("s") * _SC_CORES + lax.axis_index("c")
        tbase = wid * tok_per_w

        def start_gather(ci, par):
            ebase = TOPK * (tbase + ci * tch)
            pltpu.sync_copy(idx_hbm.at[pl.ds(ebase, TOPK * tch)], idx_v.at[par])
            return pltpu.async_copy(eo_hbm.at[idx_v.at[par]], rows_v.at[par],
                                    sems.at[par])

        cps = [start_gather(0, 0), None]
        for ci in range(nch):
            par = ci % 2
            if ci + 1 < nch:
                cps[(ci + 1) % 2] = start_gather(ci + 1, (ci + 1) % 2)
            cps[par].wait()

            @plsc.parallel_loop(0, tch)
            def _(ti):
                @plsc.parallel_loop(0, c, step=64)
                def _(cc):
                    for u in range(4):
                        r0 = rows_v[par, 2 * ti, pl.ds(cc + u * 16, 16)]
                        r1 = rows_v[par, 2 * ti + 1, pl.ds(cc + u * 16, 16)]
                        out_v[ti, pl.ds(cc + u * 16, 16)] = r0 + r1

            pltpu.sync_copy(out_v, o_hbm.at[pl.ds(tbase + ci * tch, tch)])

    return k(eo, slot_flat)


def kernel(x, router_w, fc_w, proj_w):
    b, t, c = x.shape
    n = b * t
    e, hd, _ = fc_w.shape
    cap = 2 * n * TOPK // e

    x2d = x.reshape(n, c)
    probs, w2, slot_d, counts = _router(x2d, router_w, cap)
    xg, wslot = _dispatch_sc(x2d, slot_d[:, 0], slot_d[:, 1],
                             w2[:, 0], w2[:, 1], e * cap)
    eo = _ffn(counts.reshape(e), xg, wslot, fc_w, proj_w, cap)
    out = _combine_sc(eo, slot_d.reshape(-1))
    return out.reshape(b, t, c), probs.reshape(b, t, e)
